# baseline (speedup 1.0000x reference)
import functools

import jax
import jax.numpy as jnp
from jax import lax
from jax.experimental import pallas as pl
from jax.experimental.pallas import tpu as pltpu
from jax.experimental.pallas import tpu_sc as plsc

_NC = 2
_NS = 16
_NW = _NC * _NS
_L = 16

_T = 100
_N = 4096
_B = _N * _T
_D = 64
_DW = _D // 2
_BW = _B // _NW
_C = 128
_NCHUNK = _BW // _C
_NBUF = 10
_NG = _NCHUNK // _NBUF
_SP = 512
_ROWS = 5

_RES = 5000.0
_INV_RES = 1.0 / 5000.0


def _compute_idx_chunk(frac_v, idx_v, c):
    base = c * _C
    for k in range(_C // _L):
        fr = frac_v[pl.ds(base + k * _L, _L)]
        fr = jnp.maximum(fr, _INV_RES)
        x = fr * _RES
        y = x + 0.5
        f = y.astype(jnp.int32)
        tie = f.astype(jnp.float32) == y
        odd = f & 1
        idx = f - 1 - jnp.where(tie, odd, 0)
        idx_v[c, pl.ds(k * _L, _L)] = idx


def _encoder_kernel(frac_hbm, pe_hbm, out_hbm, frac_v, idx_v, rows_v,
                    pe_sh, gsems, osems):
    sid = lax.axis_index("s")
    wid = sid * _NC + lax.axis_index("c")
    base = wid * _BW

    @pl.when(sid == 0)
    def _():
        pltpu.sync_copy(pe_hbm, pe_sh)

    pltpu.sync_copy(frac_hbm.at[pl.ds(base, _BW)], frac_v)
    plsc.subcore_barrier()

    def fire_gather(b, c):
        pltpu.async_copy(pe_sh.at[idx_v.at[c]], rows_v.at[b], gsems.at[b])

    def wait_gather(b, c):
        pltpu.make_async_copy(pe_sh.at[idx_v.at[c]], rows_v.at[b],
                              gsems.at[b]).wait()

    def fire_out(b, c):
        pltpu.async_copy(rows_v.at[b],
                         out_hbm.at[pl.ds(base + c * _C, _C)], osems.at[b])

    def wait_out(b, c):
        pltpu.make_async_copy(rows_v.at[b],
                              out_hbm.at[pl.ds(base + c * _C, _C)],
                              osems.at[b]).wait()

    _compute_idx_chunk(frac_v, idx_v, 0)
    _compute_idx_chunk(frac_v, idx_v, 1)
    fire_gather(0, 0)

    for b in range(_NBUF):
        c = b
        if b == _NBUF - 1:
            wait_out(0, 0)
        fire_gather((b + 1) % _NBUF, c + 1)
        _compute_idx_chunk(frac_v, idx_v, c + 2)
        wait_gather(b, c)
        fire_out(b, c)

    def body(g, carry):
        for b in range(_NBUF):
            c = g * _NBUF + b
            b1 = (b + 1) % _NBUF
            wait_out(b1, c + 1 - _NBUF)
            fire_gather(b1, c + 1)
            _compute_idx_chunk(frac_v, idx_v, c + 2)
            wait_gather(b, c)
            fire_out(b, c)
        return carry

    lax.fori_loop(1, _NG - 1, body, 0)

    for b in range(_NBUF):
        c = (_NG - 1) * _NBUF + b
        b1 = (b + 1) % _NBUF
        wait_out(b1, c + 1 - _NBUF)
        if c + 1 < _NCHUNK:
            fire_gather(b1, c + 1)
        if c + 2 < _NCHUNK:
            _compute_idx_chunk(frac_v, idx_v, c + 2)
        wait_gather(b, c)
        fire_out(b, c)

    for b in range(1, _NBUF):
        wait_out(b, (_NG - 1) * _NBUF + b)


def _panel_transpose_kernel(x_ref, y_ref):
    for r in range(_ROWS):
        for s in range(_N // _SP):
            blk = x_ref[pl.ds((r * (_N // _SP) + s) * _SP * _DW,
                              _SP * _DW)].reshape(_C, _C)
            t = blk.T
            lo = lax.bitcast_convert_type(t << 16, jnp.float32)
            hi = lax.bitcast_convert_type(
                t & jnp.int32(-65536), jnp.float32)
            for h in range(_SP // _C):
                b0 = s * _SP + h * _C
                y_ref[r, 0:_DW, b0:b0 + _C] = lo[h * _DW:(h + 1) * _DW, :]
                y_ref[r, _DW:_D, b0:b0 + _C] = hi[h * _DW:(h + 1) * _DW, :]


@jax.jit
def kernel(frac, pe):
    pe16 = pe.astype(jnp.bfloat16)
    pe_packed = lax.bitcast_convert_type(
        jnp.stack([pe16[:, :_DW], pe16[:, _DW:]], axis=-1), jnp.int32)
    frac_flat = (frac.T.reshape(_T, _N // _SP, 4, _C)
                 .transpose(0, 1, 3, 2).reshape(_B))
    mesh = plsc.VectorSubcoreMesh(core_axis_name="c", subcore_axis_name="s",
                                  num_cores=_NC, num_subcores=_NS)
    out_flat = pl.kernel(
        _encoder_kernel,
        out_type=jax.ShapeDtypeStruct((_B, _DW), jnp.int32),
        mesh=mesh,
        compiler_params=pltpu.CompilerParams(use_tc_tiling_on_sc=False),
        scratch_types=[
            pltpu.VMEM((_BW,), jnp.float32),
            pltpu.VMEM((_NCHUNK, _C), jnp.int32),
            pltpu.VMEM((_NBUF, _C, _DW), jnp.int32),
            pltpu.VMEM_SHARED((5000, _DW), jnp.int32),
            pltpu.SemaphoreType.DMA((_NBUF,)),
            pltpu.SemaphoreType.DMA((_NBUF,)),
        ],
    )(frac_flat, pe_packed)

    x1d = out_flat.reshape(_B * _DW)
    y = pl.pallas_call(
        _panel_transpose_kernel,
        grid=(_T // _ROWS,),
        in_specs=[pl.BlockSpec((_ROWS * _N * _DW,), lambda t: (t,))],
        out_specs=pl.BlockSpec((_ROWS, _D, _N), lambda t: (t, 0, 0)),
        out_shape=jax.ShapeDtypeStruct((_T, _D, _N), jnp.float32),
    )(x1d)
    return y.transpose(2, 0, 1)


# speedup vs baseline: 1.0188x; 1.0188x over previous
import functools

import jax
import jax.numpy as jnp
from jax import lax
from jax.experimental import pallas as pl
from jax.experimental.pallas import tpu as pltpu
from jax.experimental.pallas import tpu_sc as plsc

_NC = 2
_NS = 16
_NW = _NC * _NS
_L = 16

_T = 100
_N = 4096
_B = _N * _T
_D = 64
_DW = _D // 2
_BW = _B // _NW
_C = 128
_NCHUNK = _BW // _C
_NBUF = 10
_NG = _NCHUNK // _NBUF
_SP = 512
_ROWS = 10

_RES = 5000.0
_INV_RES = 1.0 / 5000.0


def _compute_idx_chunk(frac_v, idx_v, c):
    base = c * _C
    for k in range(_C // _L):
        fr = frac_v[pl.ds(base + k * _L, _L)]
        fr = jnp.maximum(fr, _INV_RES)
        x = fr * _RES
        y = x + 0.5
        f = y.astype(jnp.int32)
        tie = f.astype(jnp.float32) == y
        odd = f & 1
        idx = f - 1 - jnp.where(tie, odd, 0)
        idx_v[c, pl.ds(k * _L, _L)] = idx


def _encoder_kernel(frac_hbm, pe_hbm, out_hbm, frac_v, idx_v, rows_v,
                    pe_sh, gsems, osems):
    sid = lax.axis_index("s")
    wid = sid * _NC + lax.axis_index("c")
    base = wid * _BW

    @pl.when(sid == 0)
    def _():
        pltpu.sync_copy(pe_hbm, pe_sh)

    pltpu.sync_copy(frac_hbm.at[pl.ds(base, _BW)], frac_v)
    plsc.subcore_barrier()

    def fire_gather(b, c):
        pltpu.async_copy(pe_sh.at[idx_v.at[c]], rows_v.at[b], gsems.at[b])

    def wait_gather(b, c):
        pltpu.make_async_copy(pe_sh.at[idx_v.at[c]], rows_v.at[b],
                              gsems.at[b]).wait()

    def fire_out(b, c):
        pltpu.async_copy(rows_v.at[b],
                         out_hbm.at[pl.ds(base + c * _C, _C)], osems.at[b])

    def wait_out(b, c):
        pltpu.make_async_copy(rows_v.at[b],
                              out_hbm.at[pl.ds(base + c * _C, _C)],
                              osems.at[b]).wait()

    _compute_idx_chunk(frac_v, idx_v, 0)
    _compute_idx_chunk(frac_v, idx_v, 1)
    fire_gather(0, 0)

    for b in range(_NBUF):
        c = b
        if b == _NBUF - 1:
            wait_out(0, 0)
        fire_gather((b + 1) % _NBUF, c + 1)
        _compute_idx_chunk(frac_v, idx_v, c + 2)
        wait_gather(b, c)
        fire_out(b, c)

    def body(g, carry):
        for b in range(_NBUF):
            c = g * _NBUF + b
            b1 = (b + 1) % _NBUF
            wait_out(b1, c + 1 - _NBUF)
            fire_gather(b1, c + 1)
            _compute_idx_chunk(frac_v, idx_v, c + 2)
            wait_gather(b, c)
            fire_out(b, c)
        return carry

    lax.fori_loop(1, _NG - 1, body, 0)

    for b in range(_NBUF):
        c = (_NG - 1) * _NBUF + b
        b1 = (b + 1) % _NBUF
        wait_out(b1, c + 1 - _NBUF)
        if c + 1 < _NCHUNK:
            fire_gather(b1, c + 1)
        if c + 2 < _NCHUNK:
            _compute_idx_chunk(frac_v, idx_v, c + 2)
        wait_gather(b, c)
        fire_out(b, c)

    for b in range(1, _NBUF):
        wait_out(b, (_NG - 1) * _NBUF + b)


def _panel_transpose_kernel(x_ref, y_ref):
    for r in range(_ROWS):
        for s in range(_N // _SP):
            blk = x_ref[pl.ds((r * (_N // _SP) + s) * _SP * _DW,
                              _SP * _DW)].reshape(_C, _C)
            t = blk.T
            lo = lax.bitcast_convert_type(t << 16, jnp.float32)
            hi = lax.bitcast_convert_type(
                t & jnp.int32(-65536), jnp.float32)
            for h in range(_SP // _C):
                b0 = s * _SP + h * _C
                y_ref[r, 0:_DW, b0:b0 + _C] = lo[h * _DW:(h + 1) * _DW, :]
                y_ref[r, _DW:_D, b0:b0 + _C] = hi[h * _DW:(h + 1) * _DW, :]


@jax.jit
def kernel(frac, pe):
    pe16 = pe.astype(jnp.bfloat16)
    pe_packed = lax.bitcast_convert_type(
        jnp.stack([pe16[:, :_DW], pe16[:, _DW:]], axis=-1), jnp.int32)
    frac_flat = (frac.T.reshape(_T, _N // _SP, 4, _C)
                 .transpose(0, 1, 3, 2).reshape(_B))
    mesh = plsc.VectorSubcoreMesh(core_axis_name="c", subcore_axis_name="s",
                                  num_cores=_NC, num_subcores=_NS)
    out_flat = pl.kernel(
        _encoder_kernel,
        out_type=jax.ShapeDtypeStruct((_B, _DW), jnp.int32),
        mesh=mesh,
        compiler_params=pltpu.CompilerParams(use_tc_tiling_on_sc=False),
        scratch_types=[
            pltpu.VMEM((_BW,), jnp.float32),
            pltpu.VMEM((_NCHUNK, _C), jnp.int32),
            pltpu.VMEM((_NBUF, _C, _DW), jnp.int32),
            pltpu.VMEM_SHARED((5000, _DW), jnp.int32),
            pltpu.SemaphoreType.DMA((_NBUF,)),
            pltpu.SemaphoreType.DMA((_NBUF,)),
        ],
    )(frac_flat, pe_packed)

    x1d = out_flat.reshape(_B * _DW)
    y = pl.pallas_call(
        _panel_transpose_kernel,
        grid=(_T // _ROWS,),
        in_specs=[pl.BlockSpec((_ROWS * _N * _DW,), lambda t: (t,))],
        out_specs=pl.BlockSpec((_ROWS, _D, _N), lambda t: (t, 0, 0)),
        out_shape=jax.ShapeDtypeStruct((_T, _D, _N), jnp.float32),
    )(x1d)
    return y.transpose(2, 0, 1)
